# baseline (speedup 1.0000x reference)
import functools

import numpy as np
import jax
import jax.numpy as jnp
from jax import lax
from jax.experimental import pallas as pl
from jax.experimental.pallas import tpu as pltpu
from jax.experimental.pallas import tpu_sc as plsc

T = 32768
V = 32
NW = 32
ROWS = T // NW
HALO = 16
LROWS = ROWS + HALO
GROUPS = LROWS // 16


def _sc_body(emission_hbm, out_hbm, emis_v, ids_v, out_v):
    c = lax.axis_index("c")
    s = lax.axis_index("s")
    wid = s * 2 + c
    start = wid * ROWS

    off = jnp.where(wid > 0, HALO, 0)
    load_start = start - off
    pltpu.sync_copy(emission_hbm.at[pl.ds(load_start * V, LROWS * V)], emis_v)

    iota = lax.iota(jnp.int32, 16)
    cols = [(iota + v0) % V for v0 in range(V)]

    def argmax_group(g, _):
        row_base = (g * 16 + iota) * V
        cur_max = plsc.load_gather(emis_v, [row_base + cols[0]])
        cur_id = cols[0]
        for v0 in range(1, V):
            vals = plsc.load_gather(emis_v, [row_base + cols[v0]])
            gt = vals > cur_max
            tie = (vals == cur_max) & (cols[v0] < cur_id)
            better = gt | tie
            cur_max = jnp.where(better, vals, cur_max)
            cur_id = jnp.where(better, cols[v0], cur_id)
        ids_v[pl.ds(g * 16, 16)] = cur_id
        return 0

    lax.fori_loop(0, GROUPS, argmax_group, 0)

    def collapse_group(g, _):
        base = off + g * 16
        cur = plsc.load_gather(ids_v, [base + iota])
        prev_idx = base - 1 + iota
        prev_raw = plsc.load_gather(ids_v, [jnp.maximum(prev_idx, 0)])
        prev = jnp.where(prev_idx >= 0, prev_raw, -1)
        keep = (cur != prev) & (cur != 0)
        out_v[pl.ds(g * 16, 16)] = jnp.where(keep, cur, -1)
        return 0

    lax.fori_loop(0, ROWS // 16, collapse_group, 0)

    pltpu.sync_copy(out_v, out_hbm.at[pl.ds(start, ROWS)])


_ctc_sc = functools.partial(
    pl.kernel,
    out_type=jax.ShapeDtypeStruct((T,), jnp.int32),
    mesh=plsc.VectorSubcoreMesh(core_axis_name="c", subcore_axis_name="s"),
    compiler_params=pltpu.CompilerParams(
        use_tc_tiling_on_sc=False, needs_layout_passes=False),
    scratch_types=[
        pltpu.VMEM((LROWS * V,), jnp.float32),
        pltpu.VMEM((LROWS,), jnp.int32),
        pltpu.VMEM((ROWS,), jnp.int32),
    ],
)(_sc_body)


@jax.jit
def kernel(emission):
    return _ctc_sc(emission.reshape(T * V))
